# baseline (speedup 1.0000x reference)
import functools
import math

import jax
import jax.numpy as jnp
from jax import lax
from jax.experimental import pallas as pl
from jax.experimental.pallas import tpu as pltpu
from jax.experimental.pallas import tpu_sc as plsc

_L = 16
_BB = 512
_SUB = 128
_NBUF = 2


@functools.partial(jax.jit, static_argnames=("bsz", "h", "d"))
def _emb_lookup(xt, table, bsz, h, d):
    info = plsc.get_sparse_core_info()
    nc, ns = info.num_cores, info.num_subcores
    nw = nc * ns
    n_bblk = bsz // _BB
    n_hblk = nw // n_bblk
    h_per_w = h // n_hblk
    n_sub = _BB // _SUB
    scale = math.sqrt(float(d))

    mesh = plsc.VectorSubcoreMesh(core_axis_name="c", subcore_axis_name="s")

    @functools.partial(
        pl.kernel,
        mesh=mesh,
        compiler_params=pltpu.CompilerParams(
            use_tc_tiling_on_sc=False, needs_layout_passes=False
        ),
        out_type=jax.ShapeDtypeStruct((h, d, bsz), jnp.float32),
        scratch_types=[
            pltpu.VMEM((h_per_w, _BB), jnp.int32),
            pltpu.VMEM((_NBUF, _BB, d), jnp.float32),
            pltpu.VMEM((_NBUF, d, _BB), jnp.float32),
            pltpu.SemaphoreType.DMA((_NBUF,)),
            pltpu.SemaphoreType.DMA((_NBUF,)),
        ],
    )
    def k(xt_hbm, table_hbm, out_hbm, idx_v, gbuf, tbuf, gsem, ssem):
        wid = lax.axis_index("s") * nc + lax.axis_index("c")
        h0 = (wid // n_bblk) * h_per_w
        b0 = (wid % n_bblk) * _BB
        pltpu.sync_copy(xt_hbm.at[pl.ds(h0, h_per_w), pl.ds(b0, _BB)], idx_v)

        def fire_gathers(s, b):
            for q in range(n_sub):
                pltpu.async_copy(
                    table_hbm.at[idx_v.at[s, pl.ds(q * _SUB, _SUB)]],
                    gbuf.at[b, pl.ds(q * _SUB, _SUB)],
                    gsem.at[b],
                )

        def wait_gathers(b):
            for q in range(n_sub):
                pltpu.make_async_copy(
                    table_hbm.at[idx_v.at[0, pl.ds(0, _SUB)]],
                    gbuf.at[b, pl.ds(0, _SUB)],
                    gsem.at[b],
                ).wait()

        for b in range(_NBUF):
            fire_gathers(b, b)

        lane = lax.iota(jnp.int32, _L)

        def slab_body(s, carry):
            b = lax.rem(s, _NBUF)

            wait_gathers(b)

            @pl.when(s >= _NBUF)
            def _():
                pltpu.make_async_copy(
                    tbuf.at[b], out_hbm.at[0, :, pl.ds(0, _BB)], ssem.at[b]
                ).wait()

            for dd in range(d):
                col = jnp.full((_L,), dd, jnp.int32)

                @plsc.parallel_loop(0, _BB // _L, 1, unroll=4)
                def _(g):
                    rows = g * _L + lane
                    v = plsc.load_gather(gbuf.at[b], [rows, col])
                    tbuf[b, dd, pl.ds(g * _L, _L)] = v * scale

            pltpu.async_copy(
                tbuf.at[b],
                out_hbm.at[h0 + s, :, pl.ds(b0, _BB)],
                ssem.at[b],
            )

            @pl.when(s < h_per_w - _NBUF)
            def _():
                fire_gathers(s + _NBUF, b)

            return carry

        lax.fori_loop(0, h_per_w, slab_body, 0)

        for b in range(_NBUF):
            pltpu.make_async_copy(
                tbuf.at[b], out_hbm.at[0, :, pl.ds(0, _BB)], ssem.at[b]
            ).wait()

    return k(xt, table)


def kernel(x, table):
    bsz, h = x.shape
    v, d = table.shape
    info = plsc.get_sparse_core_info()
    nw = info.num_cores * info.num_subcores
    assert bsz % _BB == 0 and h % (nw // (bsz // _BB)) == 0
    xt = jnp.transpose(x.astype(jnp.int32), (1, 0))
    out = _emb_lookup(xt, table, bsz, h, d)
    return jnp.transpose(out, (2, 0, 1))


# speedup vs baseline: 1.6229x; 1.6229x over previous
import functools
import math

import jax
import jax.numpy as jnp
from jax import lax
from jax.experimental import pallas as pl
from jax.experimental.pallas import tpu as pltpu
from jax.experimental.pallas import tpu_sc as plsc

_L = 16
_BB = 512
_SUB = 128
_NBUF = 2


@functools.partial(jax.jit, static_argnames=("bsz", "h", "d"))
def _emb_lookup(xt, table, bsz, h, d):
    info = plsc.get_sparse_core_info()
    nc, ns = info.num_cores, info.num_subcores
    nw = nc * ns
    n_bblk = bsz // _BB
    n_hblk = nw // n_bblk
    h_per_w = h // n_hblk
    n_sub = _BB // _SUB
    scale = math.sqrt(float(d))

    mesh = plsc.VectorSubcoreMesh(core_axis_name="c", subcore_axis_name="s")

    @functools.partial(
        pl.kernel,
        mesh=mesh,
        compiler_params=pltpu.CompilerParams(
            use_tc_tiling_on_sc=False, needs_layout_passes=False
        ),
        out_type=jax.ShapeDtypeStruct((h, d, bsz), jnp.float32),
        scratch_types=[
            pltpu.VMEM((h_per_w, _BB), jnp.int32),
            pltpu.VMEM((_NBUF, _BB, d), jnp.float32),
            pltpu.VMEM((_NBUF, d, _BB + 1), jnp.float32),
            pltpu.SemaphoreType.DMA((_NBUF,)),
            pltpu.SemaphoreType.DMA((_NBUF,)),
        ],
    )
    def k(xt_hbm, table_hbm, out_hbm, idx_v, gbuf, tbuf, gsem, ssem):
        wid = lax.axis_index("s") * nc + lax.axis_index("c")
        h0 = (wid // n_bblk) * h_per_w
        b0 = (wid % n_bblk) * _BB
        pltpu.sync_copy(xt_hbm.at[pl.ds(h0, h_per_w), pl.ds(b0, _BB)], idx_v)

        def fire_gathers(s, b):
            for q in range(n_sub):
                pltpu.async_copy(
                    table_hbm.at[idx_v.at[s, pl.ds(q * _SUB, _SUB)]],
                    gbuf.at[b, pl.ds(q * _SUB, _SUB)],
                    gsem.at[b],
                )

        def wait_gathers(b):
            for q in range(n_sub):
                pltpu.make_async_copy(
                    table_hbm.at[idx_v.at[0, pl.ds(0, _SUB)]],
                    gbuf.at[b, pl.ds(0, _SUB)],
                    gsem.at[b],
                ).wait()

        for b in range(_NBUF):
            fire_gathers(b, b)

        lane = lax.iota(jnp.int32, _L)

        def slab_body(s, carry):
            b = lax.rem(s, _NBUF)

            wait_gathers(b)

            @pl.when(s >= _NBUF)
            def _():
                pltpu.make_async_copy(
                    tbuf.at[b, :, pl.ds(0, _BB)],
                    out_hbm.at[0, :, pl.ds(0, _BB)],
                    ssem.at[b],
                ).wait()

            @plsc.parallel_loop(0, _BB, 1, unroll=4)
            def _(r):
                rcol = jnp.full((_L,), r, jnp.int32)
                for jj in range(0, d, _L):
                    v = gbuf[b, r, pl.ds(jj, _L)]
                    plsc.store_scatter(
                        tbuf.at[b], [jj + lane, rcol], v * scale
                    )

            pltpu.async_copy(
                tbuf.at[b, :, pl.ds(0, _BB)],
                out_hbm.at[h0 + s, :, pl.ds(b0, _BB)],
                ssem.at[b],
            )

            @pl.when(s < h_per_w - _NBUF)
            def _():
                fire_gathers(s + _NBUF, b)

            return carry

        lax.fori_loop(0, h_per_w, slab_body, 0)

        for b in range(_NBUF):
            pltpu.make_async_copy(
                tbuf.at[b, :, pl.ds(0, _BB)],
                out_hbm.at[0, :, pl.ds(0, _BB)],
                ssem.at[b],
            ).wait()

    return k(xt, table)


def kernel(x, table):
    bsz, h = x.shape
    v, d = table.shape
    info = plsc.get_sparse_core_info()
    nw = info.num_cores * info.num_subcores
    assert bsz % _BB == 0 and h % (nw // (bsz // _BB)) == 0
    xt = jnp.transpose(x.astype(jnp.int32), (1, 0))
    out = _emb_lookup(xt, table, bsz, h, d)
    return jnp.transpose(out, (2, 0, 1))


# speedup vs baseline: 1.8778x; 1.1570x over previous
import functools
import math

import jax
import jax.numpy as jnp
from jax import lax
from jax.experimental import pallas as pl
from jax.experimental.pallas import tpu as pltpu
from jax.experimental.pallas import tpu_sc as plsc

_L = 16
_BB = 512
_SUB = 128
_NBUF = 2


@functools.partial(jax.jit, static_argnames=("bsz", "h", "d"))
def _emb_lookup(xt, table, bsz, h, d):
    info = plsc.get_sparse_core_info()
    nc, ns = info.num_cores, info.num_subcores
    nw = nc * ns
    n_bblk = bsz // _BB
    n_hblk = nw // n_bblk
    h_per_w = h // n_hblk
    n_sub = _BB // _SUB
    n_dg = d // 8
    n_bg = bsz // 128
    w_bg = _BB // 128
    scale = math.sqrt(float(d))

    mesh = plsc.VectorSubcoreMesh(core_axis_name="c", subcore_axis_name="s")

    @functools.partial(
        pl.kernel,
        mesh=mesh,
        compiler_params=pltpu.CompilerParams(
            use_tc_tiling_on_sc=False, needs_layout_passes=False
        ),
        out_type=jax.ShapeDtypeStruct((h, n_dg, n_bg, 8, 128), jnp.float32),
        scratch_types=[
            pltpu.VMEM((h_per_w, _BB), jnp.int32),
            pltpu.VMEM((_NBUF, _BB, d), jnp.float32),
            pltpu.VMEM((_NBUF, n_dg, w_bg, 8, 129), jnp.float32),
            pltpu.SemaphoreType.DMA((_NBUF,)),
            pltpu.SemaphoreType.DMA((_NBUF,)),
        ],
    )
    def k(xt_hbm, table_hbm, out_hbm, idx_v, gbuf, tbuf, gsem, ssem):
        wid = lax.axis_index("s") * nc + lax.axis_index("c")
        h0 = (wid // n_bblk) * h_per_w
        b0 = (wid % n_bblk) * _BB
        bg0 = (wid % n_bblk) * w_bg
        pltpu.sync_copy(xt_hbm.at[pl.ds(h0, h_per_w), pl.ds(b0, _BB)], idx_v)

        def fire_gathers(s, b):
            for q in range(n_sub):
                pltpu.async_copy(
                    table_hbm.at[idx_v.at[s, pl.ds(q * _SUB, _SUB)]],
                    gbuf.at[b, pl.ds(q * _SUB, _SUB)],
                    gsem.at[b],
                )

        def wait_gathers(b):
            for q in range(n_sub):
                pltpu.make_async_copy(
                    table_hbm.at[idx_v.at[0, pl.ds(0, _SUB)]],
                    gbuf.at[b, pl.ds(0, _SUB)],
                    gsem.at[b],
                ).wait()

        def store_src(b):
            return tbuf.at[b, :, pl.ds(0, w_bg), :, pl.ds(0, 128)]

        for b in range(_NBUF):
            fire_gathers(b, b)

        lane = lax.iota(jnp.int32, _L)

        def slab_body(s, carry):
            b = lax.rem(s, _NBUF)

            wait_gathers(b)

            @pl.when(s >= _NBUF)
            def _():
                pltpu.make_async_copy(
                    store_src(b),
                    out_hbm.at[0, :, pl.ds(0, w_bg)],
                    ssem.at[b],
                ).wait()

            @plsc.parallel_loop(0, _BB, 1, unroll=4)
            def _(r):
                bg = jnp.full((_L,), lax.shift_right_logical(r, 7), jnp.int32)
                b1 = jnp.full((_L,), lax.bitwise_and(r, 127), jnp.int32)
                for jj in range(0, d, _L):
                    dv = jj + lane
                    v = gbuf[b, r, pl.ds(jj, _L)]
                    plsc.store_scatter(
                        tbuf.at[b],
                        [lax.shift_right_logical(dv, 3), bg,
                         lax.bitwise_and(dv, 7), b1],
                        v * scale,
                    )

            pltpu.async_copy(
                store_src(b),
                out_hbm.at[h0 + s, :, pl.ds(bg0, w_bg)],
                ssem.at[b],
            )

            @pl.when(s < h_per_w - _NBUF)
            def _():
                fire_gathers(s + _NBUF, b)

            return carry

        lax.fori_loop(0, h_per_w, slab_body, 0)

        for b in range(_NBUF):
            pltpu.make_async_copy(
                store_src(b),
                out_hbm.at[0, :, pl.ds(0, w_bg)],
                ssem.at[b],
            ).wait()

    return k(xt, table)


def kernel(x, table):
    bsz, h = x.shape
    v, d = table.shape
    info = plsc.get_sparse_core_info()
    nw = info.num_cores * info.num_subcores
    assert bsz % _BB == 0 and h % (nw // (bsz // _BB)) == 0 and d % 8 == 0
    xt = jnp.transpose(x.astype(jnp.int32), (1, 0))
    out = _emb_lookup(xt, table, bsz, h, d)
    out = jnp.transpose(out, (2, 4, 0, 1, 3))
    return out.reshape(bsz, h, d)
